# baseline (speedup 1.0000x reference)
import functools

import numpy as np
import jax
import jax.numpy as jnp
from jax import lax
from jax.experimental import pallas as pl
from jax.experimental.pallas import tpu as pltpu
from jax.experimental.pallas import tpu_sc as plsc

_NW = 32


def _sc_gather(x_flat, table, bsz, seq):
    n = x_flat.shape[0]
    v, e = table.shape
    bpw = n // _NW
    rpw = bsz // _NW
    mesh = plsc.VectorSubcoreMesh(core_axis_name="c", subcore_axis_name="s")

    nck = 4
    ck = bpw // nck

    def body(idx_hbm, table_hbm, out_hbm, idx_v, b0, b1,
             gs0, gs1, ws0, ws1):
        sid = lax.axis_index("s")
        wid = sid * 2 + lax.axis_index("c")
        base = wid * bpw
        pltpu.sync_copy(idx_hbm.at[pl.ds(base, bpw)], idx_v)
        bufs = (b0, b1)
        gsems = (gs0, gs1)
        wsems = (ws0, ws1)
        gops = [None] * nck
        wops = [None] * nck
        for c in range(nck):
            b = c & 1
            if c >= 2:
                wops[c - 2].wait()
            gops[c] = pltpu.async_copy(
                table_hbm.at[idx_v.at[pl.ds(c * ck, ck)]], bufs[b], gsems[b])
            if c >= 1:
                gops[c - 1].wait()
                wops[c - 1] = pltpu.async_copy(
                    bufs[1 - b], out_hbm.at[pl.ds(base + (c - 1) * ck, ck)],
                    wsems[1 - b])
        gops[nck - 1].wait()
        wops[nck - 1] = pltpu.async_copy(
            bufs[(nck - 1) & 1], out_hbm.at[pl.ds(base + (nck - 1) * ck, ck)],
            wsems[(nck - 1) & 1])
        wops[nck - 2].wait()
        wops[nck - 1].wait()

    return pl.kernel(
        body,
        mesh=mesh,
        out_type=jax.ShapeDtypeStruct((n, e), jnp.float32),
        scratch_types=[
            pltpu.VMEM((bpw,), jnp.int32),
            pltpu.VMEM((ck, e), jnp.float32),
            pltpu.VMEM((ck, e), jnp.float32),
            pltpu.SemaphoreType.DMA,
            pltpu.SemaphoreType.DMA,
            pltpu.SemaphoreType.DMA,
            pltpu.SemaphoreType.DMA,
        ],
        compiler_params=pltpu.CompilerParams(use_tc_tiling_on_sc=False),
    )(x_flat, table)


_INV_SQRT_P0 = float(1.0 / np.sqrt(64.0))
_INV_SQRT_P1 = float(1.0 / np.sqrt(32.0))


def _pen_math(emb_ref, W0_ref, b0_ref, P0_ref, P0T_ref, Wp0_ref, bp0_ref,
              W1_ref, b1_ref, P1_ref, P1T_ref, Wp1_ref, bp1_ref,
              Wpen_ref, bpen_ref):
    f32 = jnp.float32
    bb = emb_ref.shape[0] // 25
    z0 = jnp.zeros((bb, W0_ref.shape[1]), f32) + b0_ref[...]
    for j in range(25):
        hj = emb_ref[pl.Slice(j, bb, 25), :]
        z0 = z0 + jnp.dot(hj, W0_ref[pl.ds(128 * j, 128), :],
                          preferred_element_type=f32)
    a0 = jnp.maximum(z0, 0.0)
    s0 = jnp.dot(a0, P0T_ref[...], preferred_element_type=f32) * _INV_SQRT_P0
    e0 = jnp.exp(s0 - jnp.max(s0, axis=-1, keepdims=True))
    attn0 = e0 / jnp.sum(e0, axis=-1, keepdims=True)
    recon0 = jnp.dot(attn0, P0_ref[...], preferred_element_type=f32)
    pred0 = jnp.dot(a0, Wp0_ref[...], preferred_element_type=f32) + bp0_ref[...]

    z1 = jnp.dot(recon0, W1_ref[...], preferred_element_type=f32) + b1_ref[...]
    a1 = jnp.maximum(z1, 0.0)
    s1 = jnp.dot(a1, P1T_ref[...], preferred_element_type=f32) * _INV_SQRT_P1
    e1 = jnp.exp(s1 - jnp.max(s1, axis=-1, keepdims=True))
    attn1 = e1 / jnp.sum(e1, axis=-1, keepdims=True)
    recon1 = jnp.dot(attn1, P1_ref[...], preferred_element_type=f32)
    pred1 = jnp.dot(a1, Wp1_ref[...], preferred_element_type=f32) + bp1_ref[...]

    pen = jnp.maximum(
        jnp.dot(recon1, Wpen_ref[...], preferred_element_type=f32) + bpen_ref[...], 0.0)

    err0 = jnp.mean((pred0 - pen) ** 2, axis=-1, keepdims=True)
    err1 = jnp.mean((pred1 - pen) ** 2, axis=-1, keepdims=True)
    pe8 = jnp.concatenate(
        [err0, err1, jnp.zeros((err0.shape[0], 6), f32)], axis=1)
    return pen, pe8


def _mlp_body(emb_ref, W0_ref, b0_ref, P0_ref, P0T_ref, Wp0_ref, bp0_ref,
              W1_ref, b1_ref, P1_ref, P1T_ref, Wp1_ref, bp1_ref,
              Wpen_ref, bpen_ref, pen_ref, pe_ref):
    pen, pe8 = _pen_math(emb_ref, W0_ref, b0_ref, P0_ref, P0T_ref, Wp0_ref,
                         bp0_ref, W1_ref, b1_ref, P1_ref, P1T_ref, Wp1_ref,
                         bp1_ref, Wpen_ref, bpen_ref)
    pen_ref[...] = pen
    pe_ref[...] = pe8


def _fused_body(emb_ref, W0_ref, b0_ref, P0_ref, P0T_ref, Wp0_ref, bp0_ref,
                W1_ref, b1_ref, P1_ref, P1T_ref, Wp1_ref, bp1_ref,
                Wpen_ref, bpen_ref, waug_ref,
                ot_ref, pe_ref, pen_scr):
    f32 = jnp.float32

    @pl.when(pl.program_id(0) == 0)
    def _():
        pen, pe8 = _pen_math(emb_ref, W0_ref, b0_ref, P0_ref, P0T_ref,
                             Wp0_ref, bp0_ref, W1_ref, b1_ref, P1_ref,
                             P1T_ref, Wp1_ref, bp1_ref, Wpen_ref, bpen_ref)
        pen_scr[...] = jnp.concatenate(
            [pen, jnp.ones((pen.shape[0], 1), f32)], axis=1)
        pe_ref[...] = pe8

    ot_ref[...] = lax.dot_general(
        waug_ref[...], pen_scr[...], (((0,), (1,)), ((), ())),
        preferred_element_type=f32)


def _fused_half_body(dummy_ref, emb_ref, W0_ref, b0_ref, P0_ref, P0T_ref,
                     Wp0_ref, bp0_ref, W1_ref, b1_ref, P1_ref, P1T_ref,
                     Wp1_ref, bp1_ref, Wpen_ref, bpen_ref, waug_ref,
                     ot_ref, pe_ref, pen_scr):
    del dummy_ref
    _fused_body(emb_ref, W0_ref, b0_ref, P0_ref, P0T_ref, Wp0_ref, bp0_ref,
                W1_ref, b1_ref, P1_ref, P1T_ref, Wp1_ref, bp1_ref,
                Wpen_ref, bpen_ref, waug_ref, ot_ref, pe_ref, pen_scr)


def _fused(emb128, W0, b0, P0, Wp0, bp0, W1, b1, P1, Wp1, bp1,
           W_pen, b_pen, w_aug, bsz, half, prev_out):
    hh = emb128.shape[0] * 128 // W0.shape[0]
    ka, v = w_aug.shape
    bn = 2048
    full = lambda i: (0, 0)
    in_specs = [
        pl.BlockSpec(emb128.shape, full),
        pl.BlockSpec(W0.shape, full),
        pl.BlockSpec((1, 64), full),
        pl.BlockSpec(P0.shape, full),
        pl.BlockSpec((64, 8), full),
        pl.BlockSpec(Wp0.shape, full),
        pl.BlockSpec((1, 32), full),
        pl.BlockSpec(W1.shape, full),
        pl.BlockSpec((1, 32), full),
        pl.BlockSpec(P1.shape, full),
        pl.BlockSpec((32, 8), full),
        pl.BlockSpec(Wp1.shape, full),
        pl.BlockSpec((1, 32), full),
        pl.BlockSpec(W_pen.shape, full),
        pl.BlockSpec((1, 32), full),
        pl.BlockSpec((ka, bn), lambda i: (0, i)),
    ]
    args = [emb128, W0, b0.reshape(1, -1), P0, P0.T, Wp0, bp0.reshape(1, -1),
            W1, b1.reshape(1, -1), P1, P1.T, Wp1, bp1.reshape(1, -1),
            W_pen, b_pen.reshape(1, -1), w_aug]
    body = _fused_body
    aliases = {}
    if prev_out is not None:
        in_specs = [pl.BlockSpec((8, 128), full)] + in_specs
        args = [prev_out] + args
        body = _fused_half_body
        aliases = {0: 0}
    return pl.pallas_call(
        body,
        grid=(pl.cdiv(v, bn),),
        in_specs=in_specs,
        out_specs=[
            pl.BlockSpec((bn, hh), lambda i, h=half: (i, h)),
            pl.BlockSpec((hh, 8), full),
        ],
        out_shape=[
            jax.ShapeDtypeStruct((v, bsz), jnp.float32),
            jax.ShapeDtypeStruct((hh, 8), jnp.float32),
        ],
        scratch_shapes=[pltpu.VMEM((hh, ka), jnp.float32)],
        input_output_aliases=aliases,
    )(*args)


def _mlp(emb128, W0, b0, P0, Wp0, bp0, W1, b1, P1, Wp1, bp1, W_pen, b_pen):
    bsz = emb128.shape[0] * 128 // W0.shape[0]
    bb = 256
    grid = bsz // bb
    full = lambda i: (0, 0)
    return pl.pallas_call(
        _mlp_body,
        grid=(grid,),
        in_specs=[
            pl.BlockSpec((bb * 25, 128), lambda i: (i, 0)),
            pl.BlockSpec(W0.shape, full),
            pl.BlockSpec((1, 64), full),
            pl.BlockSpec(P0.shape, full),
            pl.BlockSpec((64, 8), full),
            pl.BlockSpec(Wp0.shape, full),
            pl.BlockSpec((1, 32), full),
            pl.BlockSpec(W1.shape, full),
            pl.BlockSpec((1, 32), full),
            pl.BlockSpec(P1.shape, full),
            pl.BlockSpec((32, 8), full),
            pl.BlockSpec(Wp1.shape, full),
            pl.BlockSpec((1, 32), full),
            pl.BlockSpec(W_pen.shape, full),
            pl.BlockSpec((1, 32), full),
        ],
        out_specs=[
            pl.BlockSpec((bb, 32), lambda i: (i, 0)),
            pl.BlockSpec((bb, 8), lambda i: (i, 0)),
        ],
        out_shape=[
            jax.ShapeDtypeStruct((bsz, 32), jnp.float32),
            jax.ShapeDtypeStruct((bsz, 8), jnp.float32),
        ],
    )(emb128, W0, b0.reshape(1, -1), P0, P0.T, Wp0, bp0.reshape(1, -1),
      W1, b1.reshape(1, -1), P1, P1.T, Wp1, bp1.reshape(1, -1),
      W_pen, b_pen.reshape(1, -1))


def _proj_body(pen_ref, w_ref, b_ref, o_ref):
    o_ref[...] = jnp.dot(pen_ref[...], w_ref[...],
                         preferred_element_type=jnp.float32) + b_ref[...]


def _proj(pen, W_out, b_out):
    bsz, k = pen.shape
    v = W_out.shape[1]
    bn = 4096
    return pl.pallas_call(
        _proj_body,
        grid=(pl.cdiv(v, bn),),
        in_specs=[
            pl.BlockSpec((bsz, k), lambda i: (0, 0)),
            pl.BlockSpec((k, bn), lambda i: (0, i)),
            pl.BlockSpec((1, bn), lambda i: (0, i)),
        ],
        out_specs=pl.BlockSpec((bsz, bn), lambda i: (0, i)),
        out_shape=jax.ShapeDtypeStruct((bsz, v), jnp.float32),
    )(pen, W_out, b_out.reshape(1, -1))


def kernel(x, emb_table, W0, b0, P0, Wp0, bp0, W1, b1, P1, Wp1, bp1,
           W_pen, b_pen, W_out, b_out):
    bsz, seq = x.shape
    e = emb_table.shape[1]
    hb = bsz // 2
    w_aug = jnp.concatenate([W_out, b_out[None, :]], axis=0)
    x1 = x[:hb].reshape(hb * seq).astype(jnp.int32)
    x2 = x[hb:].reshape(hb * seq).astype(jnp.int32)
    rows1 = _sc_gather(x1, emb_table, hb, seq)
    rows2 = _sc_gather(x2, emb_table, hb, seq)
    emb1 = rows1.reshape(hb * seq * e // 128, 128)
    emb2 = rows2.reshape(hb * seq * e // 128, 128)
    out1, pe1 = _fused(emb1, W0, b0, P0, Wp0, bp0, W1, b1, P1, Wp1, bp1,
                       W_pen, b_pen, w_aug, bsz, 0, None)
    out2, pe2 = _fused(emb2, W0, b0, P0, Wp0, bp0, W1, b1, P1, Wp1, bp1,
                       W_pen, b_pen, w_aug, bsz, 1, out1)
    output = out2.T
    pred_errors = jnp.concatenate([pe1[:, :2].T, pe2[:, :2].T], axis=1)
    return (output, pred_errors)


# speedup vs baseline: 1.1910x; 1.1910x over previous
import numpy as np
import jax
import jax.numpy as jnp
from jax import lax
from jax.experimental import pallas as pl
from jax.experimental.pallas import tpu as pltpu
from jax.experimental.pallas import tpu_sc as plsc

_NW = 32


def _sc_gather(x_flat, table, bsz, seq):
    del bsz, seq
    n = x_flat.shape[0]
    e = table.shape[1]
    bpw = n // _NW
    mesh = plsc.VectorSubcoreMesh(core_axis_name="c", subcore_axis_name="s")

    nck = 4
    ck = bpw // nck

    def body(idx_hbm, table_hbm, out_hbm, idx_v, b0, b1,
             gs0, gs1, ws0, ws1):
        sid = lax.axis_index("s")
        wid = sid * 2 + lax.axis_index("c")
        base = wid * bpw
        pltpu.sync_copy(idx_hbm.at[pl.ds(base, bpw)], idx_v)
        bufs = (b0, b1)
        gsems = (gs0, gs1)
        wsems = (ws0, ws1)
        gops = [None] * nck
        wops = [None] * nck
        for c in range(nck):
            b = c & 1
            if c >= 2:
                wops[c - 2].wait()
            gops[c] = pltpu.async_copy(
                table_hbm.at[idx_v.at[pl.ds(c * ck, ck)]], bufs[b], gsems[b])
            if c >= 1:
                gops[c - 1].wait()
                wops[c - 1] = pltpu.async_copy(
                    bufs[1 - b], out_hbm.at[pl.ds(base + (c - 1) * ck, ck)],
                    wsems[1 - b])
        gops[nck - 1].wait()
        wops[nck - 1] = pltpu.async_copy(
            bufs[(nck - 1) & 1], out_hbm.at[pl.ds(base + (nck - 1) * ck, ck)],
            wsems[(nck - 1) & 1])
        wops[nck - 2].wait()
        wops[nck - 1].wait()

    return pl.kernel(
        body,
        mesh=mesh,
        out_type=jax.ShapeDtypeStruct((n, e), jnp.float32),
        scratch_types=[
            pltpu.VMEM((bpw,), jnp.int32),
            pltpu.VMEM((ck, e), jnp.float32),
            pltpu.VMEM((ck, e), jnp.float32),
            pltpu.SemaphoreType.DMA,
            pltpu.SemaphoreType.DMA,
            pltpu.SemaphoreType.DMA,
            pltpu.SemaphoreType.DMA,
        ],
        compiler_params=pltpu.CompilerParams(use_tc_tiling_on_sc=False),
    )(x_flat, table)


_INV_SQRT_P0 = float(1.0 / np.sqrt(64.0))
_INV_SQRT_P1 = float(1.0 / np.sqrt(32.0))


def _pen_math(emb_ref, W0_ref, b0_ref, P0_ref, P0T_ref, Wp0_ref, bp0_ref,
              W1_ref, b1_ref, P1_ref, P1T_ref, Wp1_ref, bp1_ref,
              Wpen_ref, bpen_ref):
    f32 = jnp.float32
    bb = emb_ref.shape[0] // 25
    z0 = jnp.zeros((bb, W0_ref.shape[1]), f32) + b0_ref[...]
    for j in range(25):
        hj = emb_ref[pl.Slice(j, bb, 25), :]
        z0 = z0 + jnp.dot(hj, W0_ref[pl.ds(128 * j, 128), :],
                          preferred_element_type=f32)
    a0 = jnp.maximum(z0, 0.0)
    s0 = jnp.dot(a0, P0T_ref[...], preferred_element_type=f32) * _INV_SQRT_P0
    e0 = jnp.exp(s0 - jnp.max(s0, axis=-1, keepdims=True))
    attn0 = e0 / jnp.sum(e0, axis=-1, keepdims=True)
    recon0 = jnp.dot(attn0, P0_ref[...], preferred_element_type=f32)
    pred0 = jnp.dot(a0, Wp0_ref[...], preferred_element_type=f32) + bp0_ref[...]

    z1 = jnp.dot(recon0, W1_ref[...], preferred_element_type=f32) + b1_ref[...]
    a1 = jnp.maximum(z1, 0.0)
    s1 = jnp.dot(a1, P1T_ref[...], preferred_element_type=f32) * _INV_SQRT_P1
    e1 = jnp.exp(s1 - jnp.max(s1, axis=-1, keepdims=True))
    attn1 = e1 / jnp.sum(e1, axis=-1, keepdims=True)
    recon1 = jnp.dot(attn1, P1_ref[...], preferred_element_type=f32)
    pred1 = jnp.dot(a1, Wp1_ref[...], preferred_element_type=f32) + bp1_ref[...]

    pen = jnp.maximum(
        jnp.dot(recon1, Wpen_ref[...], preferred_element_type=f32) + bpen_ref[...], 0.0)

    err0 = jnp.mean((pred0 - pen) ** 2, axis=-1, keepdims=True)
    err1 = jnp.mean((pred1 - pen) ** 2, axis=-1, keepdims=True)
    pe8 = jnp.concatenate(
        [err0, err1, jnp.zeros((err0.shape[0], 6), f32)], axis=1)
    return pen, pe8


def _fused_body(emb_ref, W0_ref, b0_ref, P0_ref, P0T_ref, Wp0_ref, bp0_ref,
                W1_ref, b1_ref, P1_ref, P1T_ref, Wp1_ref, bp1_ref,
                Wpen_ref, bpen_ref, waug_ref,
                ot_ref, pe_ref, pen_scr):
    f32 = jnp.float32

    @pl.when(pl.program_id(0) == 0)
    def _():
        pen, pe8 = _pen_math(emb_ref, W0_ref, b0_ref, P0_ref, P0T_ref,
                             Wp0_ref, bp0_ref, W1_ref, b1_ref, P1_ref,
                             P1T_ref, Wp1_ref, bp1_ref, Wpen_ref, bpen_ref)
        pen_scr[...] = jnp.concatenate(
            [pen, jnp.ones((pen.shape[0], 1), f32)], axis=1)
        pe_ref[...] = pe8

    ot_ref[...] = lax.dot_general(
        waug_ref[...], pen_scr[...], (((0,), (1,)), ((), ())),
        preferred_element_type=f32)


def _fused(emb128, W0, b0, P0, Wp0, bp0, W1, b1, P1, Wp1, bp1,
           W_pen, b_pen, w_aug):
    bsz = emb128.shape[0] * 128 // W0.shape[0]
    ka, v = w_aug.shape
    bn = 2048
    full = lambda i: (0, 0)
    return pl.pallas_call(
        _fused_body,
        grid=(pl.cdiv(v, bn),),
        in_specs=[
            pl.BlockSpec(emb128.shape, full),
            pl.BlockSpec(W0.shape, full),
            pl.BlockSpec((1, 64), full),
            pl.BlockSpec(P0.shape, full),
            pl.BlockSpec((64, 8), full),
            pl.BlockSpec(Wp0.shape, full),
            pl.BlockSpec((1, 32), full),
            pl.BlockSpec(W1.shape, full),
            pl.BlockSpec((1, 32), full),
            pl.BlockSpec(P1.shape, full),
            pl.BlockSpec((32, 8), full),
            pl.BlockSpec(Wp1.shape, full),
            pl.BlockSpec((1, 32), full),
            pl.BlockSpec(W_pen.shape, full),
            pl.BlockSpec((1, 32), full),
            pl.BlockSpec((ka, bn), lambda i: (0, i)),
        ],
        out_specs=[
            pl.BlockSpec((bn, bsz), lambda i: (i, 0)),
            pl.BlockSpec((bsz, 8), full),
        ],
        out_shape=[
            jax.ShapeDtypeStruct((v, bsz), jnp.float32),
            jax.ShapeDtypeStruct((bsz, 8), jnp.float32),
        ],
        scratch_shapes=[pltpu.VMEM((bsz, ka), jnp.float32)],
    )(emb128, W0, b0.reshape(1, -1), P0, P0.T, Wp0, bp0.reshape(1, -1),
      W1, b1.reshape(1, -1), P1, P1.T, Wp1, bp1.reshape(1, -1),
      W_pen, b_pen.reshape(1, -1), w_aug)


def kernel(x, emb_table, W0, b0, P0, Wp0, bp0, W1, b1, P1, Wp1, bp1,
           W_pen, b_pen, W_out, b_out):
    bsz, seq = x.shape
    e = emb_table.shape[1]
    x_flat = x.reshape(bsz * seq).astype(jnp.int32)
    rows = _sc_gather(x_flat, emb_table, bsz, seq)
    emb128 = rows.reshape(bsz * seq * e // 128, 128)
    w_aug = jnp.concatenate([W_out, b_out[None, :]], axis=0)
    out_t, pe = _fused(emb128, W0, b0, P0, Wp0, bp0, W1, b1, P1, Wp1, bp1,
                       W_pen, b_pen, w_aug)
    output = out_t.T
    pred_errors = pe[:, :2].T
    return (output, pred_errors)
